# baseline (speedup 1.0000x reference)
import functools

import jax
import jax.numpy as jnp
from jax import lax
from jax.experimental import pallas as pl
from jax.experimental.pallas import tpu as pltpu
from jax.experimental.pallas import tpu_sc as plsc

D = 64
NC = 2
NS = 16
NW = NC * NS
G = 128
GPC = 4
CHUNK = G * GPC


def _embed_call(B, V):
    b_per_w = B // NW
    n_gather = b_per_w // G
    n_chunks = b_per_w // CHUNK
    mesh = plsc.VectorSubcoreMesh(
        core_axis_name="c", subcore_axis_name="s",
        num_cores=NC, num_subcores=NS)

    @functools.partial(
        pl.kernel,
        mesh=mesh,
        compiler_params=pltpu.CompilerParams(use_tc_tiling_on_sc=False),
        out_type=jax.ShapeDtypeStruct((B, D), jnp.float32),
        scratch_types=[
            pltpu.VMEM((n_gather, G), jnp.int32),
            pltpu.VMEM((CHUNK, D), jnp.float32),
            pltpu.VMEM((CHUNK, D), jnp.float32),
            pltpu.SemaphoreType.DMA,
            pltpu.SemaphoreType.DMA,
            pltpu.SemaphoreType.DMA,
            pltpu.SemaphoreType.DMA,
        ],
    )
    def k(idx_hbm, table_hbm, out_hbm, idx_v, buf0, buf1,
          sem_g0, sem_g1, sem_s0, sem_s1):
        wid = lax.axis_index("s") * NC + lax.axis_index("c")
        base = wid * b_per_w
        pltpu.sync_copy(idx_hbm.at[wid], idx_v)

        bufs = (buf0, buf1)
        sems_g = (sem_g0, sem_g1)
        sems_s = (sem_s0, sem_s1)

        def issue_gathers(c, p):
            for j in range(GPC):
                pltpu.async_copy(
                    table_hbm.at[idx_v.at[c * GPC + j]],
                    bufs[p].at[pl.ds(j * G, G)],
                    sems_g[p])

        def wait_gathers(p):
            pltpu.make_async_copy(
                table_hbm.at[pl.ds(0, CHUNK)], bufs[p], sems_g[p]).wait()

        def issue_store(c, p):
            pltpu.async_copy(
                bufs[p], out_hbm.at[pl.ds(base + c * CHUNK, CHUNK)],
                sems_s[p])

        def wait_store(p):
            pltpu.make_async_copy(
                bufs[p], out_hbm.at[pl.ds(0, CHUNK)], sems_s[p]).wait()

        issue_gathers(0, 0)
        issue_gathers(1, 1)

        def body(i, _):
            c0 = 2 * i
            for p in range(2):
                c = c0 + p
                wait_gathers(p)
                issue_store(c, p)
                wait_store(p)

                @pl.when(c + 2 < n_chunks)
                def _():
                    issue_gathers(c + 2, p)
            return 0

        lax.fori_loop(0, n_chunks // 2, body, 0)

    return k


def kernel(x, weight):
    orig_shape = x.shape
    idx = x.reshape(-1).astype(jnp.int32)
    B = idx.shape[0]
    idx3 = idx.reshape(NW, (B // NW) // G, G)
    out = _embed_call(B, weight.shape[0])(idx3, weight)
    return out.reshape(*orig_shape, D)
